# baseline (speedup 1.0000x reference)
import functools

import jax
import jax.numpy as jnp
from jax.experimental import pallas as pl
from jax.experimental.pallas import tpu as pltpu


def _se_fused_kernel(x_ref, w1t_ref, w2t_ref, o_hbm, scratch, sem, *, inv_hw):
    i = pl.program_id(0)
    n = pl.num_programs(0)
    slot = jax.lax.rem(i, 2)

    @pl.when(i >= 2)
    def _():
        pltpu.make_async_copy(scratch.at[slot], o_hbm.at[i - 2], sem.at[slot]).wait()

    y = jnp.sum(x_ref[...], axis=-1) * inv_hw
    hdn = jnp.maximum(
        jnp.dot(y, w1t_ref[...], preferred_element_type=jnp.float32), 0.0)
    s = jax.nn.sigmoid(
        jnp.dot(hdn, w2t_ref[...], preferred_element_type=jnp.float32))
    scratch[slot] = x_ref[0] * s[0, :, None]
    pltpu.make_async_copy(scratch.at[slot], o_hbm.at[i], sem.at[slot]).start(priority=1)

    @pl.when(i == n - 1)
    def _():
        pltpu.make_async_copy(scratch.at[slot], o_hbm.at[i], sem.at[slot]).wait()
        pltpu.make_async_copy(
            scratch.at[1 - slot], o_hbm.at[i - 1], sem.at[1 - slot]).wait()


def kernel(x_nchw, w1, w2):
    b, c, h, w = x_nchw.shape
    hw = h * w
    cr = w1.shape[0]
    hwp = (hw + 127) // 128 * 128

    x = x_nchw.reshape(b, c, hw).astype(jnp.float32)
    xp = jnp.pad(x, ((0, 0), (0, 0), (0, hwp - hw)))
    w1t = w1.T.astype(jnp.float32)
    w2t = w2.T.astype(jnp.float32)

    out = pl.pallas_call(
        functools.partial(_se_fused_kernel, inv_hw=1.0 / float(hw)),
        out_shape=jax.ShapeDtypeStruct((b, c, hwp), jnp.float32),
        grid=(b,),
        in_specs=[
            pl.BlockSpec((1, c, hwp), lambda i: (i, 0, 0)),
            pl.BlockSpec((c, cr), lambda i: (0, 0)),
            pl.BlockSpec((cr, c), lambda i: (0, 0)),
        ],
        out_specs=pl.BlockSpec(memory_space=pl.ANY),
        scratch_shapes=[
            pltpu.VMEM((2, c, hwp), jnp.float32),
            pltpu.SemaphoreType.DMA((2,)),
        ],
        compiler_params=pltpu.CompilerParams(
            dimension_semantics=("arbitrary",),
            vmem_limit_bytes=48 * 1024 * 1024,
            allow_input_fusion=[True, False, False],
        ),
        cost_estimate=pl.CostEstimate(
            flops=int(2 * b * c * hw + 4 * b * c * cr),
            transcendentals=int(b * c),
            bytes_accessed=int(2 * b * c * hw * 4),
        ),
    )(xp, w1t, w2t)

    return out[:, :, :hw].reshape(b, c, h, w).astype(x_nchw.dtype)


# speedup vs baseline: 1.2772x; 1.2772x over previous
import functools

import jax
import jax.numpy as jnp
from jax.experimental import pallas as pl
from jax.experimental.pallas import tpu as pltpu


def _se_fused_kernel(x_ref, w1t_ref, w2t_ref, o_ref, *, inv_hw):
    y = jnp.sum(x_ref[...], axis=-1) * inv_hw
    hdn = jnp.maximum(
        jnp.dot(y, w1t_ref[...], preferred_element_type=jnp.float32), 0.0)
    s = jax.nn.sigmoid(
        jnp.dot(hdn, w2t_ref[...], preferred_element_type=jnp.float32))
    o_ref[...] = x_ref[...] * s[:, :, None]


def kernel(x_nchw, w1, w2):
    b, c, h, w = x_nchw.shape
    hw = h * w
    cr = w1.shape[0]
    hwp = (hw + 127) // 128 * 128

    x = x_nchw.reshape(b, c, hw).astype(jnp.float32)
    xp = jnp.pad(x, ((0, 0), (0, 0), (0, hwp - hw)))
    w1t = w1.T.astype(jnp.float32)
    w2t = w2.T.astype(jnp.float32)

    out = pl.pallas_call(
        functools.partial(_se_fused_kernel, inv_hw=1.0 / float(hw)),
        out_shape=jax.ShapeDtypeStruct((b, c, hwp), jnp.float32),
        grid=(b,),
        in_specs=[
            pl.BlockSpec((1, c, hwp), lambda i: (i, 0, 0)),
            pl.BlockSpec((c, cr), lambda i: (0, 0)),
            pl.BlockSpec((cr, c), lambda i: (0, 0)),
        ],
        out_specs=pl.BlockSpec((1, c, hwp), lambda i: (i, 0, 0)),
        compiler_params=pltpu.CompilerParams(
            dimension_semantics=("arbitrary",),
            vmem_limit_bytes=48 * 1024 * 1024,
            allow_input_fusion=[True, False, False],
        ),
        cost_estimate=pl.CostEstimate(
            flops=int(2 * b * c * hw + 4 * b * c * cr),
            transcendentals=int(b * c),
            bytes_accessed=int(2 * b * c * hw * 4),
        ),
    )(xp, w1t, w2t)

    return out[:, :, :hw].reshape(b, c, h, w).astype(x_nchw.dtype)


# speedup vs baseline: 1.2777x; 1.0005x over previous
import functools

import jax
import jax.numpy as jnp
from jax.experimental import pallas as pl
from jax.experimental.pallas import tpu as pltpu


def _se_fused_kernel(x_ref, w1t_ref, w2t_ref, o_ref, *, inv_hw):
    y = jnp.sum(x_ref[...], axis=-1) * inv_hw
    hdn = jnp.maximum(
        jnp.dot(y, w1t_ref[...], preferred_element_type=jnp.float32), 0.0)
    s = jax.nn.sigmoid(
        jnp.dot(hdn, w2t_ref[...], preferred_element_type=jnp.float32))
    o_ref[...] = x_ref[...] * s[:, :, None]


def kernel(x_nchw, w1, w2):
    b, c, h, w = x_nchw.shape
    hw = h * w
    cr = w1.shape[0]
    hwp = (hw + 127) // 128 * 128

    x = x_nchw.reshape(b, c, hw).astype(jnp.float32)
    xp = jnp.pad(x, ((0, 0), (0, 0), (0, hwp - hw)))
    w1t = w1.T.astype(jnp.float32)
    w2t = w2.T.astype(jnp.float32)

    out = pl.pallas_call(
        functools.partial(_se_fused_kernel, inv_hw=1.0 / float(hw)),
        out_shape=jax.ShapeDtypeStruct((b, c, hwp), jnp.float32),
        grid=(b,),
        in_specs=[
            pl.BlockSpec((1, c, hwp), lambda i: (i, 0, 0)),
            pl.BlockSpec((c, cr), lambda i: (0, 0)),
            pl.BlockSpec((cr, c), lambda i: (0, 0)),
        ],
        out_specs=pl.BlockSpec((1, c, hwp), lambda i: (i, 0, 0)),
        compiler_params=pltpu.CompilerParams(
            dimension_semantics=("parallel",),
            vmem_limit_bytes=48 * 1024 * 1024,
            allow_input_fusion=[True, True, True],
        ),
        cost_estimate=pl.CostEstimate(
            flops=int(2 * b * c * hw + 4 * b * c * cr),
            transcendentals=int(b * c),
            bytes_accessed=int(2 * b * c * hw * 4),
        ),
    )(xp, w1t, w2t)

    return out[:, :, :hw].reshape(b, c, h, w).astype(x_nchw.dtype)


# speedup vs baseline: 1.2954x; 1.0138x over previous
import functools

import jax
import jax.numpy as jnp
from jax.experimental import pallas as pl
from jax.experimental.pallas import tpu as pltpu


def _se_fused_kernel(x_ref, w1t_ref, w2t_ref, o_ref, *, inv_hw):
    y = jnp.sum(x_ref[...], axis=-1) * inv_hw
    hdn = jnp.maximum(
        jnp.dot(y, w1t_ref[...], preferred_element_type=jnp.float32), 0.0)
    s = jax.nn.sigmoid(
        jnp.dot(hdn, w2t_ref[...], preferred_element_type=jnp.float32))
    o_ref[...] = x_ref[...] * s[:, :, None]


def kernel(x_nchw, w1, w2):
    b, c, h, w = x_nchw.shape
    hw = h * w
    cr = w1.shape[0]
    hwp = (hw + 127) // 128 * 128

    x = x_nchw.reshape(b, c, hw).astype(jnp.float32)
    xp = jnp.pad(x, ((0, 0), (0, 0), (0, hwp - hw)))
    w1t = w1.T.astype(jnp.float32)
    w2t = w2.T.astype(jnp.float32)

    out = pl.pallas_call(
        functools.partial(_se_fused_kernel, inv_hw=1.0 / float(hw)),
        out_shape=jax.ShapeDtypeStruct((b, c, hwp), jnp.float32),
        grid=(b // 2,),
        in_specs=[
            pl.BlockSpec((2, c, hwp), lambda i: (i, 0, 0)),
            pl.BlockSpec((c, cr), lambda i: (0, 0)),
            pl.BlockSpec((cr, c), lambda i: (0, 0)),
        ],
        out_specs=pl.BlockSpec((2, c, hwp), lambda i: (i, 0, 0)),
        compiler_params=pltpu.CompilerParams(
            dimension_semantics=("parallel",),
            vmem_limit_bytes=48 * 1024 * 1024,
            allow_input_fusion=[True, True, True],
        ),
        cost_estimate=pl.CostEstimate(
            flops=int(2 * b * c * hw + 4 * b * c * cr),
            transcendentals=int(b * c),
            bytes_accessed=int(2 * b * c * hw * 4),
        ),
    )(xp, w1t, w2t)

    return out[:, :, :hw].reshape(b, c, h, w).astype(x_nchw.dtype)
